# Initial kernel scaffold; baseline (speedup 1.0000x reference)
import jax
import jax.numpy as jnp
from jax.experimental import pallas as pl


def kernel(h, edge_index, edge_weight, gamma1, beta1, W_l, b_l, W_r, b_r, W_e, att, bias_conv, W1, b1, gamma_fc, beta_fc, W2, b2):
    raise NotImplementedError("write your pallas kernel here")


# speedup vs baseline: 14.0214x; 14.0214x over previous
import functools

import jax
import jax.numpy as jnp
from jax import lax
from jax.experimental import pallas as pl
from jax.experimental.pallas import tpu as pltpu
from jax.experimental.pallas import tpu_sc as plsc

NC = 2
NS = 16
NW = NC * NS
CH = 128

H = 2
C = 128


def _sc_mesh():
    return plsc.VectorSubcoreMesh(
        core_axis_name="c", subcore_axis_name="s", num_cores=NC, num_subcores=NS
    )


def _gather_rows(table, src2d, dst2d, e):
    nchunk = src2d.shape[0]
    nloop = (nchunk + NW - 1) // NW

    @functools.partial(
        pl.kernel,
        out_type=(
            jax.ShapeDtypeStruct((e, 16), jnp.float32),
            jax.ShapeDtypeStruct((e, 16), jnp.float32),
        ),
        mesh=_sc_mesh(),
        compiler_params=pltpu.CompilerParams(use_tc_tiling_on_sc=False),
        scratch_types=[
            pltpu.VMEM((CH,), jnp.int32),
            pltpu.VMEM((CH, 16), jnp.float32),
            pltpu.SemaphoreType.DMA,
        ],
    )
    def k(table_h, src_h, dst_h, xs_h, xd_h, idx_v, rows_v, sem):
        wid = lax.axis_index("s") * NC + lax.axis_index("c")

        def body(j, carry):
            r = j * NW + wid

            @pl.when(r < nchunk)
            def _():
                pltpu.sync_copy(src_h.at[r], idx_v)
                pltpu.async_copy(table_h.at[idx_v], rows_v, sem).wait()
                pltpu.sync_copy(rows_v, xs_h.at[pl.ds(r * CH, CH)])
                pltpu.sync_copy(dst_h.at[r], idx_v)
                pltpu.async_copy(table_h.at[idx_v], rows_v, sem).wait()
                pltpu.sync_copy(rows_v, xd_h.at[pl.ds(r * CH, CH)])

            return carry

        lax.fori_loop(0, nloop, body, 0)

    return k(table, src2d, dst2d)


def _scatter_add(payload, dst2d, n):
    nchunk = dst2d.shape[0]
    nloop = (nchunk + NW - 1) // NW
    stripe = n // NS

    @functools.partial(
        pl.kernel,
        out_type=jax.ShapeDtypeStruct((NC, n, 16), jnp.float32),
        mesh=_sc_mesh(),
        compiler_params=pltpu.CompilerParams(use_tc_tiling_on_sc=False),
        scratch_types=[
            pltpu.VMEM_SHARED((n, 16), jnp.float32),
            pltpu.VMEM((1, CH), jnp.int32),
            pltpu.VMEM((CH, 16), jnp.float32),
            pltpu.VMEM((stripe, 16), jnp.float32),
        ],
    )
    def k(payload_h, dst_h, out_h, acc_s, idx_v, rows_v, zbuf_v):
        cid = lax.axis_index("c")
        sid = lax.axis_index("s")
        wid = sid * NC + cid

        z16 = jnp.zeros((16,), jnp.float32)

        def zbody(i, carry):
            zbuf_v[i] = z16
            return carry

        lax.fori_loop(0, stripe, zbody, 0)
        pltpu.sync_copy(zbuf_v, acc_s.at[pl.ds(sid * stripe, stripe)])
        plsc.subcore_barrier()

        def body(j, carry):
            r = j * NW + wid

            @pl.when(r < nchunk)
            def _():
                pltpu.sync_copy(dst_h.at[r], idx_v.at[0])
                pltpu.sync_copy(payload_h.at[pl.ds(r * CH, CH)], rows_v)
                pltpu.sync_copy(rows_v, acc_s.at[idx_v.at[0]], add=True)

            return carry

        lax.fori_loop(0, nloop, body, 0)
        plsc.subcore_barrier()

        pltpu.sync_copy(acc_s.at[pl.ds(sid * stripe, stripe)], zbuf_v)
        pltpu.sync_copy(zbuf_v, out_h.at[cid, pl.ds(sid * stripe, stripe)])

    return k(payload, dst2d)


def _edge_math(xs, xd, ew, Ml, Mr, We_row, const_row, abd, g16, beta16, e):
    BE = 3200
    grid = (e // BE,)
    full = lambda shape: pl.BlockSpec(shape, lambda i: (0, 0))

    def body(xs_r, xd_r, ew_r, ml_r, mr_r, we_r, cr_r, abd_r, g_r, b_r, out_r):
        xs_b = xs_r[...]
        z = (
            jnp.dot(xd_r[...], mr_r[...], preferred_element_type=jnp.float32, precision=lax.Precision.HIGHEST)
            + jnp.dot(xs_b, ml_r[...], preferred_element_type=jnp.float32, precision=lax.Precision.HIGHEST)
            + ew_r[...] * we_r[...]
            + cr_r[...]
        )
        z = jnp.where(z > 0, z, 0.2 * z)
        al = jnp.dot(z, abd_r[...], preferred_element_type=jnp.float32, precision=lax.Precision.HIGHEST)
        ex = jnp.exp(al)
        ex0 = ex[:, 0:1]
        ex1 = ex[:, 1:2]
        xsb = xs_b * g_r[...] + b_r[...]
        rolled = jnp.concatenate([xsb[:, 11:], xsb[:, :11]], axis=1)
        lane = lax.broadcasted_iota(jnp.int32, (BE, 16), 1)
        p = (
            ex0 * jnp.where(lane < 5, xsb, 0.0)
            + ex1 * jnp.where((lane >= 5) & (lane < 10), rolled, 0.0)
            + jnp.where(lane == 10, ex0, 0.0)
            + jnp.where(lane == 11, ex1, 0.0)
            + jnp.where(lane == 12, 1.0, 0.0)
        )
        out_r[...] = p

    return pl.pallas_call(
        body,
        grid=grid,
        in_specs=[
            pl.BlockSpec((BE, 16), lambda i: (i, 0)),
            pl.BlockSpec((BE, 16), lambda i: (i, 0)),
            pl.BlockSpec((BE, 1), lambda i: (i, 0)),
            full((16, H * C)),
            full((16, H * C)),
            full((1, H * C)),
            full((1, H * C)),
            full((H * C, H)),
            full((1, 16)),
            full((1, 16)),
        ],
        out_specs=pl.BlockSpec((BE, 16), lambda i: (i, 0)),
        out_shape=jax.ShapeDtypeStruct((e, 16), jnp.float32),
    )(xs, xd, ew, Ml, Mr, We_row, const_row, abd, g16, beta16)


def _node_math(acc0, acc1, G0, G1, bcv, W1p, c1, W2, b2row, n, dout):
    BN = 2000
    grid = (n // BN,)
    full = lambda shape: pl.BlockSpec(shape, lambda i: (0, 0))

    def body(a0_r, a1_r, g0_r, g1_r, bcv_r, w1_r, c1_r, w2_r, b2_r, y_r):
        a = a0_r[...] + a1_r[...]
        s0 = jnp.dot(a, g0_r[...], preferred_element_type=jnp.float32, precision=lax.Precision.HIGHEST)
        s1 = jnp.dot(a, g1_r[...], preferred_element_type=jnp.float32, precision=lax.Precision.HIGHEST)
        d0 = a[:, 10:11]
        d1 = a[:, 11:12]
        cnt = a[:, 12:13]
        oc = (s0 / (d0 + 1e-16) + s1 / (d1 + 1e-16)) * 0.5
        oc = oc / jnp.maximum(cnt, 1.0) + bcv_r[...]
        h2 = jnp.dot(oc, w1_r[...], preferred_element_type=jnp.float32, precision=lax.Precision.HIGHEST) + c1_r[...]
        h2 = jnp.where(h2 > 0, h2, 0.01 * h2)
        y_r[...] = jnp.dot(h2, w2_r[...], preferred_element_type=jnp.float32, precision=lax.Precision.HIGHEST) + b2_r[...]

    return pl.pallas_call(
        body,
        grid=grid,
        in_specs=[
            pl.BlockSpec((BN, 16), lambda i: (i, 0)),
            pl.BlockSpec((BN, 16), lambda i: (i, 0)),
            full((16, C)),
            full((16, C)),
            full((1, C)),
            full((C, C)),
            full((1, C)),
            full((C, dout)),
            full((1, dout)),
        ],
        out_specs=pl.BlockSpec((BN, dout), lambda i: (i, 0)),
        out_shape=jax.ShapeDtypeStruct((n, dout), jnp.float32),
    )(acc0, acc1, G0, G1, bcv, W1p, c1, W2, b2row)


def kernel(h, edge_index, edge_weight, gamma1, beta1, W_l, b_l, W_r, b_r, W_e,
           att, bias_conv, W1, b1, gamma_fc, beta_fc, W2, b2):
    eps = 1e-5
    n, din = h.shape
    e = edge_index.shape[1]
    dout = W2.shape[1]

    src = edge_index[0].astype(jnp.int32)
    dst = edge_index[1].astype(jnp.int32)
    src2d = src.reshape(e // CH, CH)
    dst2d = dst.reshape(e // CH, CH)

    g = gamma1 / jnp.sqrt(1.0 + eps)
    h16 = jnp.pad(h, ((0, 0), (0, 16 - din)))
    Ml = jnp.pad(g[:, None] * W_l, ((0, 16 - din), (0, 0)))
    Mr = jnp.pad(g[:, None] * W_r, ((0, 16 - din), (0, 0)))
    const_row = (b_l + b_r + beta1 @ (W_l + W_r))[None, :]
    We_row = W_e[0][None, :]
    abd = jnp.zeros((H * C, H), jnp.float32)
    abd = abd.at[:C, 0].set(att[0]).at[C:, 1].set(att[1])
    g16 = jnp.pad(g, (0, 16 - din))[None, :]
    beta16 = jnp.pad(beta1, (0, 16 - din))[None, :]
    G0 = jnp.zeros((16, C), jnp.float32).at[:din].set(W_l[:, :C]).at[10].set(b_l[:C])
    G1 = jnp.zeros((16, C), jnp.float32).at[din:2 * din].set(W_l[:, C:]).at[11].set(b_l[C:])
    gfc = gamma_fc / jnp.sqrt(1.0 + eps)
    W1p = W1 * gfc[None, :]
    c1 = (b1 * gfc + beta_fc)[None, :]
    bcv = bias_conv[None, :]
    b2row = b2[None, :]

    xs, xd = _gather_rows(h16, src2d, dst2d, e)
    payload = _edge_math(
        xs, xd, edge_weight, Ml, Mr, We_row, const_row, abd, g16, beta16, e
    )
    accs = _scatter_add(payload, dst2d, n)
    y = _node_math(accs[0], accs[1], G0, G1, bcv, W1p, c1, W2, b2row, n, dout)
    return y
